# baseline (speedup 1.0000x reference)
import functools

import jax
import jax.numpy as jnp
from jax import lax
from jax.experimental import pallas as pl
from jax.experimental.pallas import tpu as pltpu
from jax.experimental.pallas import tpu_sc as plsc

LATENT = 64
_NC = 2
_NS = 16
_NW = _NC * _NS
_L = 16
_CHB = 32


def _sc_gather_pair(src, tgt, table):
    b = src.shape[0]
    bpw = b // _NW
    nchunk = bpw // _CHB
    ntile = table.shape[0] // 8

    mesh = plsc.VectorSubcoreMesh(core_axis_name="c", subcore_axis_name="s")

    @functools.partial(
        pl.kernel,
        out_type=jax.ShapeDtypeStruct((b, 2 * LATENT), jnp.float32),
        mesh=mesh,
        scratch_types=[
            pltpu.VMEM((2 * bpw,), jnp.int32),
            pltpu.VMEM((2 * _CHB, 8, LATENT), jnp.float32),
            pltpu.VMEM((_CHB, 2 * LATENT), jnp.float32),
            pltpu.SemaphoreType.DMA,
            pltpu.SemaphoreType.DMA,
        ],
        compiler_params=pltpu.CompilerParams(needs_layout_passes=False),
    )
    def gather_kernel(src_hbm, tgt_hbm, table_hbm, out_hbm, idx_v, tiles_v,
                      rows_v, gsem, osem):
        tbl3 = table_hbm.reshape(ntile, 8, LATENT)
        wid = lax.axis_index("s") * _NC + lax.axis_index("c")
        base = wid * bpw
        pltpu.sync_copy(src_hbm.at[pl.ds(base, bpw)], idx_v.at[pl.ds(0, bpw)])
        pltpu.sync_copy(tgt_hbm.at[pl.ds(base, bpw)],
                        idx_v.at[pl.ds(bpw, bpw)])

        lanes = lax.iota(jnp.int32, _L)

        def chunk_body(c, carry):
            for h in range(2):
                for g16 in range(_CHB // _L):
                    v16 = idx_v[pl.ds(h * bpw + c * _CHB + g16 * _L, _L)]
                    for lane in range(_L):
                        s = jnp.sum(jnp.where(lanes == lane, v16, 0))
                        tq = lax.shift_right_logical(s, 3)
                        k = h * _CHB + g16 * _L + lane
                        pltpu.make_async_copy(
                            tbl3.at[tq], tiles_v.at[k], gsem).start()
            pltpu.make_async_copy(
                tbl3.at[pl.ds(0, 2 * _CHB)], tiles_v, gsem).wait()

            for h in range(2):
                for g16 in range(_CHB // _L):
                    v16 = idx_v[pl.ds(h * bpw + c * _CHB + g16 * _L, _L)]
                    r_vec = lax.bitwise_and(v16, 7)
                    g_vec = lanes + (g16 * _L)
                    k_vec = g_vec + (h * _CHB)
                    for col in range(LATENT):
                        cv = jnp.full((_L,), col, jnp.int32)
                        vals = plsc.load_gather(tiles_v, [k_vec, r_vec, cv])
                        xv = jnp.full((_L,), col + LATENT * h, jnp.int32)
                        plsc.store_scatter(rows_v, [g_vec, xv], vals)
            pltpu.sync_copy(rows_v, out_hbm.at[pl.ds(base + c * _CHB, _CHB)])
            return carry

        lax.fori_loop(0, nchunk, chunk_body, 0)

    return gather_kernel(src, tgt, table)


def _mlp_body(st_ref, w1_ref, b1_ref, w2_ref, b2_ref, o_ref):
    st = st_ref[...]
    e = st[:, :LATENT] * st[:, LATENT:]
    h = jnp.dot(e, w1_ref[...], preferred_element_type=jnp.float32)
    h = jnp.maximum(h + b1_ref[...], 0.0)
    z = jnp.sum(h * w2_ref[...], axis=1, keepdims=True) + b2_ref[...]
    o_ref[...] = jax.nn.sigmoid(z)


def kernel(src, tgt, table, W1, b1, W2, b2):
    B = src.shape[0]
    staged = _sc_gather_pair(src, tgt, table)

    blk = 2048
    nblk = B // blk
    out = pl.pallas_call(
        _mlp_body,
        grid=(nblk,),
        in_specs=[
            pl.BlockSpec((blk, 2 * LATENT), lambda i: (i, 0)),
            pl.BlockSpec((LATENT, LATENT), lambda i: (0, 0)),
            pl.BlockSpec((1, LATENT), lambda i: (0, 0)),
            pl.BlockSpec((1, LATENT), lambda i: (0, 0)),
            pl.BlockSpec((1, 1), lambda i: (0, 0)),
        ],
        out_specs=pl.BlockSpec((blk, 1), lambda i: (i, 0)),
        out_shape=jax.ShapeDtypeStruct((B, 1), jnp.float32),
    )(staged, W1, b1.reshape(1, LATENT), W2.reshape(1, LATENT),
      b2.reshape(1, 1))
    return out
